# baseline (speedup 1.0000x reference)
import functools

import jax
import jax.numpy as jnp
from jax import lax
from jax.experimental import pallas as pl
from jax.experimental.pallas import tpu as pltpu
from jax.experimental.pallas import tpu_sc as plsc

NC = 2
NS = 16
NW = NC * NS
LANES = 16
CHUNK = 128


@functools.lru_cache(maxsize=None)
def _build_transpose(H, R, V):
    NVB = (V + 127) // 128
    VB = NVB * 32
    NG = H * NVB
    NST = NG // 4
    TPW = (NST + NW - 1) // NW
    TPW = TPW + (-TPW) % 4

    mesh = plsc.VectorSubcoreMesh(core_axis_name="c", subcore_axis_name="s")

    @functools.partial(
        pl.kernel,
        mesh=mesh,
        out_type=jax.ShapeDtypeStruct((H * VB, 128), jnp.float32),
        scratch_types=[
            pltpu.VMEM((4, 4, R, 128), jnp.float32),
            pltpu.VMEM((2, 128, 128), jnp.float32),
            pltpu.SemaphoreType.DMA,
            pltpu.SemaphoreType.DMA,
            pltpu.SemaphoreType.DMA,
            pltpu.SemaphoreType.DMA,
            pltpu.SemaphoreType.DMA,
            pltpu.SemaphoreType.DMA,
        ],
        compiler_params=pltpu.CompilerParams(needs_layout_passes=False),
    )
    def tr_kernel(ft_hbm, fp_hbm, inb, outb,
                  isem0, isem1, isem2, isem3, osem0, osem1):
        cid = lax.axis_index("c")
        sid = lax.axis_index("s")
        wid = cid * NS + sid
        isems = (isem0, isem1, isem2, isem3)
        osems = (osem0, osem1)
        iota = lax.broadcasted_iota(jnp.int32, (LANES,), 0)
        rlo = [iota, iota + LANES]

        def fire_in(t, slot):
            task = wid + t * NW
            task = jnp.where(task < NST, task, 0)
            for j in range(4):
                g = task * 4 + j
                m = g // NVB
                vb = g - m * NVB
                pltpu.async_copy(
                    ft_hbm.at[m, :, pl.ds(vb * 128, 128)],
                    inb.at[slot, j], isems[slot])

        def wait_in(slot):
            for j in range(4):
                pltpu.make_async_copy(
                    ft_hbm.at[0, :, pl.ds(0, 128)], inb.at[slot, j],
                    isems[slot]).wait()

        def transpose_block(islot, oslot):
            for j in range(4):
                def row_body(v4, _):
                    for rr in range(4):
                        vr = v4 * 4 + rr
                        c4 = jnp.broadcast_to(vr * 4, (LANES,))
                        for c in range(8):
                            vals = plsc.load_gather(
                                inb.at[islot, j], [rlo[c % 2], c4 + c // 2])
                            outb[oslot, j * 32 + vr,
                                 pl.ds(c * LANES, LANES)] = vals
                    return 0
                lax.fori_loop(0, 8, row_body, 0)

        def fire_out(t, slot):
            task = wid + t * NW
            task = jnp.where(task < NST, task, 0)
            pltpu.async_copy(
                outb.at[slot],
                fp_hbm.at[pl.ds(task * 128, 128)],
                osems[slot])

        def wait_out(slot):
            pltpu.make_async_copy(
                outb.at[slot], fp_hbm.at[pl.ds(0, 128)], osems[slot]).wait()

        for p in range(4):
            fire_in(p, p)

        def step2(g2, _):
            t0 = g2 * 4
            for tt in range(4):
                t = t0 + tt
                oslot = tt % 2
                wait_in(tt)

                @pl.when(g2 * 2 + (tt // 2) > 0)
                def _():
                    wait_out(oslot)

                transpose_block(tt, oslot)
                fire_out(t, oslot)

                @pl.when(t + 4 < TPW)
                def _():
                    fire_in(t + 4, tt)

            return 0

        lax.fori_loop(0, TPW // 4, step2, 0)
        wait_out(0)
        wait_out(1)

    return tr_kernel


@functools.lru_cache(maxsize=None)
def _build_gather(B, H, R, VB):
    BPW = B // NW
    NCH = BPW // CHUNK
    NIDX = H * NCH

    mesh = plsc.VectorSubcoreMesh(core_axis_name="c", subcore_axis_name="s")

    @functools.partial(
        pl.kernel,
        mesh=mesh,
        out_type=jax.ShapeDtypeStruct((B,), jnp.float32),
        scratch_types=[
            pltpu.VMEM((NIDX, CHUNK), jnp.int32),
            pltpu.VMEM((NIDX, CHUNK), jnp.int32),
            pltpu.VMEM((2, CHUNK, 128), jnp.float32),
            pltpu.VMEM((BPW, R), jnp.float32),
            pltpu.VMEM((BPW,), jnp.float32),
            pltpu.VMEM((3, LANES), jnp.float32),
            pltpu.SemaphoreType.DMA,
            pltpu.SemaphoreType.DMA,
        ],
        compiler_params=pltpu.CompilerParams(needs_layout_passes=False),
    )
    def cp_kernel(fp_hbm, idx_hbm, q_hbm, wb_hbm, out_hbm,
                  idx_v, q_v, buf, prod, outv, wb_v, sem0, sem1):
        cid = lax.axis_index("c")
        sid = lax.axis_index("s")
        wid = cid * NS + sid
        sems = (sem0, sem1)

        pltpu.sync_copy(idx_hbm.at[pl.ds(wid * NIDX, NIDX)], idx_v)
        pltpu.sync_copy(q_hbm.at[pl.ds(wid * NIDX, NIDX)], q_v)
        pltpu.sync_copy(wb_hbm, wb_v)
        w0 = wb_v[0]
        w1 = wb_v[1]
        bias_vec = wb_v[2]

        def init_body(i, _):
            prod[i, pl.ds(0, LANES)] = w0
            prod[i, pl.ds(LANES, LANES)] = w1
            return 0
        lax.fori_loop(0, BPW, init_body, 0)

        def fire(t, slot):
            return pltpu.async_copy(
                fp_hbm.at[idx_v.at[t]], buf.at[slot], sems[slot])

        def wait_in(slot):
            pltpu.make_async_copy(
                fp_hbm.at[idx_v.at[0]], buf.at[slot], sems[slot]).wait()

        NT = H * NCH
        fire(0, 0)
        fire(1, 1)

        def step(g, _):
            for tt in range(2):
                t = g * 2 + tt
                wait_in(tt)
                base = (t % NCH) * CHUNK

                def mul_body(g2, _):
                    qvec = q_v[t, pl.ds(g2 * LANES, LANES)]
                    for k in range(LANES):
                        off = qvec[k]
                        i = g2 * LANES + k
                        row = base + i
                        prod[row, pl.ds(0, LANES)] = (
                            prod[row, pl.ds(0, LANES)]
                            * buf[tt, i, pl.ds(off, LANES)])
                        prod[row, pl.ds(LANES, LANES)] = (
                            prod[row, pl.ds(LANES, LANES)]
                            * buf[tt, i, pl.ds(off + LANES, LANES)])
                    return 0
                lax.fori_loop(0, CHUNK // LANES, mul_body, 0)

                @pl.when(t + 2 < NT)
                def _():
                    fire(t + 2, tt)

            return 0

        lax.fori_loop(0, NT // 2, step, 0)

        iota = lax.broadcasted_iota(jnp.int32, (LANES,), 0)

        def red_body(g, _):
            rows = g * LANES + iota
            acc = bias_vec
            for j in range(R):
                col = jnp.full((LANES,), j, dtype=jnp.int32)
                acc = acc + plsc.load_gather(prod, [rows, col])
            outv[pl.ds(g * LANES, LANES)] = acc
            return 0
        lax.fori_loop(0, BPW // LANES, red_body, 0)

        pltpu.sync_copy(outv, out_hbm.at[pl.ds(wid * BPW, BPW)])

    return cp_kernel


def kernel(coords, factors, weights, bias):
    H, V, R = factors.shape
    B = coords.shape[0]
    NVB = (V + 127) // 128
    VB = NVB * 32

    ft = jnp.transpose(factors, (0, 2, 1))
    fp = _build_transpose(H, R, V)(ft)

    cp_kernel = _build_gather(B, H, R, VB)

    cf = coords.astype(jnp.int32)
    marange = jnp.arange(H, dtype=jnp.int32)[None, :]
    grow = marange * VB + cf // 4
    qoff = (cf % 4) * R
    BPW = B // NW
    NCH = BPW // CHUNK

    def pack(a):
        return (a.reshape(NW, NCH, CHUNK, H)
                .transpose(0, 3, 1, 2)
                .reshape(NW * H * NCH, CHUNK))

    idx_arr = pack(grow)
    q_arr = pack(qoff)

    wb = jnp.concatenate([
        weights.astype(jnp.float32),
        jnp.broadcast_to(bias.astype(jnp.float32), (LANES,)),
    ]).reshape(3, LANES)

    return cp_kernel(fp, idx_arr, q_arr, wb)


# speedup vs baseline: 5.0562x; 5.0562x over previous
import functools

import jax
import jax.numpy as jnp
from jax import lax
from jax.experimental import pallas as pl
from jax.experimental.pallas import tpu as pltpu
from jax.experimental.pallas import tpu_sc as plsc

NC = 2
NS = 16
NW = NC * NS
LANES = 16
CHUNK = 128


@functools.lru_cache(maxsize=None)
def _build_transpose(H, R, V):
    NVB = (V + 127) // 128
    VB = NVB * 32
    NG = H * NVB
    NST = NG // 4
    TPW = (NST + NW - 1) // NW
    TPW = TPW + (-TPW) % 4

    mesh = plsc.VectorSubcoreMesh(core_axis_name="c", subcore_axis_name="s")

    @functools.partial(
        pl.kernel,
        mesh=mesh,
        out_type=jax.ShapeDtypeStruct((H * VB, 128), jnp.float32),
        scratch_types=[
            pltpu.VMEM((4, 4, R, 128), jnp.float32),
            pltpu.VMEM((2, 128, 128), jnp.float32),
            pltpu.SemaphoreType.DMA,
            pltpu.SemaphoreType.DMA,
            pltpu.SemaphoreType.DMA,
            pltpu.SemaphoreType.DMA,
            pltpu.SemaphoreType.DMA,
            pltpu.SemaphoreType.DMA,
        ],
        compiler_params=pltpu.CompilerParams(needs_layout_passes=False),
    )
    def tr_kernel(ft_hbm, fp_hbm, inb, outb,
                  isem0, isem1, isem2, isem3, osem0, osem1):
        cid = lax.axis_index("c")
        sid = lax.axis_index("s")
        wid = cid * NS + sid
        isems = (isem0, isem1, isem2, isem3)
        osems = (osem0, osem1)
        iota = lax.broadcasted_iota(jnp.int32, (LANES,), 0)
        rlo = [iota, iota + LANES]

        def fire_in(t, slot):
            task = wid + t * NW
            task = jnp.where(task < NST, task, 0)
            for j in range(4):
                g = task * 4 + j
                m = g // NVB
                vb = g - m * NVB
                pltpu.async_copy(
                    ft_hbm.at[m, :, pl.ds(vb * 128, 128)],
                    inb.at[slot, j], isems[slot])

        def wait_in(slot):
            for j in range(4):
                pltpu.make_async_copy(
                    ft_hbm.at[0, :, pl.ds(0, 128)], inb.at[slot, j],
                    isems[slot]).wait()

        def transpose_block(islot, oslot):
            for j in range(4):
                @functools.partial(plsc.parallel_loop, 0, 8, unroll=4)
                def row_body(v4):
                    for rr in range(4):
                        vr = v4 * 4 + rr
                        c4 = jnp.broadcast_to(vr * 4, (LANES,))
                        for c in range(8):
                            vals = plsc.load_gather(
                                inb.at[islot, j], [rlo[c % 2], c4 + c // 2])
                            outb[oslot, j * 32 + vr,
                                 pl.ds(c * LANES, LANES)] = vals

        def fire_out(t, slot):
            task = wid + t * NW
            task = jnp.where(task < NST, task, 0)
            pltpu.async_copy(
                outb.at[slot],
                fp_hbm.at[pl.ds(task * 128, 128)],
                osems[slot])

        def wait_out(slot):
            pltpu.make_async_copy(
                outb.at[slot], fp_hbm.at[pl.ds(0, 128)], osems[slot]).wait()

        for p in range(4):
            fire_in(p, p)

        def step2(g2, _):
            t0 = g2 * 4
            for tt in range(4):
                t = t0 + tt
                oslot = tt % 2
                wait_in(tt)

                @pl.when(g2 * 2 + (tt // 2) > 0)
                def _():
                    wait_out(oslot)

                transpose_block(tt, oslot)
                fire_out(t, oslot)

                @pl.when(t + 4 < TPW)
                def _():
                    fire_in(t + 4, tt)

            return 0

        lax.fori_loop(0, TPW // 4, step2, 0)
        wait_out(0)
        wait_out(1)

    return tr_kernel


@functools.lru_cache(maxsize=None)
def _build_gather(B, H, R, VB):
    BPW = B // NW
    NCH = BPW // CHUNK
    NIDX = H * NCH

    mesh = plsc.VectorSubcoreMesh(core_axis_name="c", subcore_axis_name="s")

    @functools.partial(
        pl.kernel,
        mesh=mesh,
        out_type=jax.ShapeDtypeStruct((B,), jnp.float32),
        scratch_types=[
            pltpu.VMEM((NIDX, CHUNK), jnp.int32),
            pltpu.VMEM((NIDX, CHUNK), jnp.int32),
            pltpu.VMEM((2, CHUNK, 128), jnp.float32),
            pltpu.VMEM((BPW, R), jnp.float32),
            pltpu.VMEM((BPW,), jnp.float32),
            pltpu.VMEM((3, LANES), jnp.float32),
            pltpu.SemaphoreType.DMA,
            pltpu.SemaphoreType.DMA,
        ],
        compiler_params=pltpu.CompilerParams(needs_layout_passes=False),
    )
    def cp_kernel(fp_hbm, idx_hbm, q_hbm, wb_hbm, out_hbm,
                  idx_v, q_v, buf, prod, outv, wb_v, sem0, sem1):
        cid = lax.axis_index("c")
        sid = lax.axis_index("s")
        wid = cid * NS + sid
        sems = (sem0, sem1)

        pltpu.sync_copy(idx_hbm.at[pl.ds(wid * NIDX, NIDX)], idx_v)
        pltpu.sync_copy(q_hbm.at[pl.ds(wid * NIDX, NIDX)], q_v)
        pltpu.sync_copy(wb_hbm, wb_v)
        w0 = wb_v[0]
        w1 = wb_v[1]
        bias_vec = wb_v[2]

        def init_body(i, _):
            prod[i, pl.ds(0, LANES)] = w0
            prod[i, pl.ds(LANES, LANES)] = w1
            return 0
        lax.fori_loop(0, BPW, init_body, 0)

        def fire(t, slot):
            return pltpu.async_copy(
                fp_hbm.at[idx_v.at[t]], buf.at[slot], sems[slot])

        def wait_in(slot):
            pltpu.make_async_copy(
                fp_hbm.at[idx_v.at[0]], buf.at[slot], sems[slot]).wait()

        NT = H * NCH
        fire(0, 0)
        fire(1, 1)

        def step(g, _):
            for tt in range(2):
                t = g * 2 + tt
                wait_in(tt)
                base = (t % NCH) * CHUNK

                def mul_body(g2, _):
                    qvec = q_v[t, pl.ds(g2 * LANES, LANES)]
                    for k in range(LANES):
                        off = qvec[k]
                        i = g2 * LANES + k
                        row = base + i
                        prod[row, pl.ds(0, LANES)] = (
                            prod[row, pl.ds(0, LANES)]
                            * buf[tt, i, pl.ds(off, LANES)])
                        prod[row, pl.ds(LANES, LANES)] = (
                            prod[row, pl.ds(LANES, LANES)]
                            * buf[tt, i, pl.ds(off + LANES, LANES)])
                    return 0
                lax.fori_loop(0, CHUNK // LANES, mul_body, 0)

                @pl.when(t + 2 < NT)
                def _():
                    fire(t + 2, tt)

            return 0

        lax.fori_loop(0, NT // 2, step, 0)

        iota = lax.broadcasted_iota(jnp.int32, (LANES,), 0)

        def red_body(g, _):
            rows = g * LANES + iota
            acc = bias_vec
            for j in range(R):
                col = jnp.full((LANES,), j, dtype=jnp.int32)
                acc = acc + plsc.load_gather(prod, [rows, col])
            outv[pl.ds(g * LANES, LANES)] = acc
            return 0
        lax.fori_loop(0, BPW // LANES, red_body, 0)

        pltpu.sync_copy(outv, out_hbm.at[pl.ds(wid * BPW, BPW)])

    return cp_kernel


def kernel(coords, factors, weights, bias):
    H, V, R = factors.shape
    B = coords.shape[0]
    NVB = (V + 127) // 128
    VB = NVB * 32

    ft = jnp.transpose(factors, (0, 2, 1))
    fp = _build_transpose(H, R, V)(ft)

    cp_kernel = _build_gather(B, H, R, VB)

    cf = coords.astype(jnp.int32)
    marange = jnp.arange(H, dtype=jnp.int32)[None, :]
    grow = marange * VB + cf // 4
    qoff = (cf % 4) * R
    BPW = B // NW
    NCH = BPW // CHUNK

    def pack(a):
        return (a.reshape(NW, NCH, CHUNK, H)
                .transpose(0, 3, 1, 2)
                .reshape(NW * H * NCH, CHUNK))

    idx_arr = pack(grow)
    q_arr = pack(qoff)

    wb = jnp.concatenate([
        weights.astype(jnp.float32),
        jnp.broadcast_to(bias.astype(jnp.float32), (LANES,)),
    ]).reshape(3, LANES)

    return cp_kernel(fp, idx_arr, q_arr, wb)
